# baseline (speedup 1.0000x reference)
import functools

import jax
import jax.numpy as jnp
from jax import lax
from jax.experimental import pallas as pl
from jax.experimental.pallas import tpu as pltpu
from jax.experimental.pallas import tpu_sc as plsc

HIDDEN = 64
BATCH = 16384

_NW = 32
_B_PER_W = BATCH // _NW
_CHUNK = 128
_N_CHUNKS = _B_PER_W // _CHUNK

_mesh = plsc.VectorSubcoreMesh(core_axis_name="c", subcore_axis_name="s")


@functools.partial(
    pl.kernel,
    mesh=_mesh,
    out_type=jax.ShapeDtypeStruct((BATCH, HIDDEN), jnp.float32),
    scratch_types=[
        pltpu.VMEM((_N_CHUNKS, _CHUNK), jnp.int32),
        pltpu.VMEM((_N_CHUNKS, _CHUNK), jnp.int32),
        pltpu.VMEM((2, _CHUNK, 2 * HIDDEN), jnp.float32),
        pltpu.VMEM((_B_PER_W, HIDDEN), jnp.float32),
        pltpu.SemaphoreType.DMA,
    ],
)
def _gather_kernel(idx_hbm, pairs_hbm, out_hbm, idx_v, jbuf, chunkb, out_v,
                   sem):
    wid = lax.axis_index("s") * 2 + lax.axis_index("c")
    base = wid * _B_PER_W
    pltpu.sync_copy(idx_hbm.at[wid], idx_v)
    for c in range(_N_CHUNKS):
        for t in range(_CHUNK // 16):
            iv = idx_v[c, pl.ds(t * 16, 16)]
            jbuf[c, pl.ds(t * 16, 16)] = lax.shift_right_logical(iv, 1)
    for c in range(2):
        pltpu.async_copy(pairs_hbm.at[jbuf.at[c]], chunkb.at[c], sem)

    def extract16(c, b, t, carry):
        ov = lax.shift_left(
            lax.bitwise_and(idx_v[c, pl.ds(t * 16, 16)], 1), 6
        )
        for l in range(16):
            o = ov[l]
            row = t * 16 + l
            for t2 in range(HIDDEN // 16):
                out_v[c * _CHUNK + row, pl.ds(t2 * 16, 16)] = chunkb[
                    b, row, pl.ds(o + t2 * 16, 16)
                ]
        return carry

    for c in range(_N_CHUNKS):
        b = c % 2
        pltpu.make_async_copy(
            pairs_hbm.at[pl.ds(0, _CHUNK)], chunkb.at[b], sem
        ).wait()
        lax.fori_loop(0, _CHUNK // 16, functools.partial(extract16, c, b), 0)
        if c + 2 < _N_CHUNKS:
            pltpu.async_copy(pairs_hbm.at[jbuf.at[c + 2]], chunkb.at[b], sem)
    pltpu.sync_copy(out_v, out_hbm.at[pl.ds(base, _B_PER_W)])


def kernel(user_id, table):
    idx = user_id.astype(jnp.int32).reshape(_NW, _N_CHUNKS, _CHUNK)
    pairs = table.reshape(table.shape[0] // 2, 2 * HIDDEN)
    return _gather_kernel(idx, pairs)


# speedup vs baseline: 2.8536x; 2.8536x over previous
import functools

import jax
import jax.numpy as jnp
from jax import lax
from jax.experimental import pallas as pl
from jax.experimental.pallas import tpu as pltpu
from jax.experimental.pallas import tpu_sc as plsc

HIDDEN = 64
BATCH = 16384
USERS = 1000000

_NW = 32
_CW = 512
_SPAN = 31744
_TAIL_LO = 31 * _SPAN + 31 * _CW
_TRASH = 63

_mesh = plsc.VectorSubcoreMesh(core_axis_name="c", subcore_axis_name="s")
_iota = lambda: lax.iota(jnp.int32, 16)


def _scal(buf, i):
    blk = lax.shift_left(lax.shift_right_logical(i, 4), 4)
    v = buf[pl.ds(blk, 16)]
    return jnp.take(v, jnp.full((16,), lax.bitwise_and(i, 15), jnp.int32))[0]


def _sortseg(cid):
    it = _iota()
    ks, vs = plsc.sort_key_val(cid, it)
    prev = jnp.take(ks, jnp.maximum(it - 1, 0))
    seg_start = (it == 0) | (ks != prev)
    seg_base = plsc.cummax(jnp.where(seg_start, it, 0))
    rank = it - seg_base
    nxt = jnp.take(seg_start.astype(jnp.int32), jnp.minimum(it + 1, 15))
    seg_last = (it == 15) | (nxt == 1)
    return ks, vs, rank, seg_last


@functools.partial(
    pl.kernel,
    mesh=_mesh,
    out_type=jax.ShapeDtypeStruct((BATCH, HIDDEN), jnp.float32),
    scratch_types=[
        pltpu.VMEM((BATCH,), jnp.int32),
        pltpu.VMEM((BATCH + 64,), jnp.int32),
        pltpu.VMEM((BATCH + 64,), jnp.int32),
        pltpu.VMEM((64,), jnp.int32),
        pltpu.VMEM((64,), jnp.int32),
        pltpu.VMEM((64,), jnp.int32),
        pltpu.VMEM((2, HIDDEN, _CW), jnp.float32),
        pltpu.VMEM((32, HIDDEN), jnp.float32),
        pltpu.SemaphoreType.DMA,
        pltpu.SemaphoreType.DMA,
        pltpu.SemaphoreType.DMA,
    ],
    compiler_params=pltpu.CompilerParams(needs_layout_passes=False),
)
def _scan_kernel(idx_hbm, tableT_hbm, tail_hbm, out_hbm, idx_s, rbuf, ibuf,
                 cnt, cur, off, chunkb, stage, semA, semB, wsem):
    wid = lax.axis_index("s") * 2 + lax.axis_index("c")
    lo = wid * _SPAN
    span = jnp.where(wid == 31, USERS - 31 * _SPAN, _SPAN)
    n_main = jnp.where(wid == 31, 31, 62)
    it = _iota()
    zeros = jnp.zeros((16,), jnp.int32)

    pltpu.async_copy(tableT_hbm.at[:, pl.ds(lo, _CW)], chunkb.at[0], semA)
    pltpu.async_copy(tableT_hbm.at[:, pl.ds(lo + _CW, _CW)], chunkb.at[1],
                     semB)
    pltpu.sync_copy(idx_hbm, idx_s)
    for q in range(4):
        cnt[pl.ds(q * 16, 16)] = zeros

    def classify(g):
        iv = idx_s[pl.ds(g * 16, 16)]
        rel = iv - lo
        m = (rel >= 0) & (rel < span)
        cid = jnp.where(m, lax.shift_right_logical(rel, 9), _TRASH)
        return rel, m, cid

    def count_body(g, carry):
        rel, m, cid = classify(g)
        npos = plsc.all_reduce_population_count(m)

        @pl.when(npos[0] > 0)
        def _():
            ks, vs, rank, seg_last = _sortseg(cid)
            plsc.addupdate_scatter(cnt, [ks], rank + 1,
                                   mask=seg_last & (ks != _TRASH))
        return carry

    lax.fori_loop(0, BATCH // 16, count_body, 0)

    carry = jnp.int32(0)
    for q in range(4):
        cv = cnt[pl.ds(q * 16, 16)]
        inc = plsc.cumsum(cv)
        ex = inc - cv + carry
        cur[pl.ds(q * 16, 16)] = ex
        off[pl.ds(q * 16, 16)] = ex
        carry = carry + inc[15]

    def place_body(g, carry):
        rel, m, cid = classify(g)
        npos = plsc.all_reduce_population_count(m)

        @pl.when(npos[0] > 0)
        def _():
            ks, vs, rank, seg_last = _sortseg(cid)
            valid = ks != _TRASH
            slots = plsc.load_gather(cur, [ks]) + rank
            slots = jnp.minimum(slots, BATCH + 48)
            plsc.store_scatter(rbuf, [slots], jnp.take(rel, vs), mask=valid)
            plsc.store_scatter(ibuf, [slots], g * 16 + vs, mask=valid)
            plsc.addupdate_scatter(cur, [ks], rank + 1,
                                   mask=seg_last & valid)
        return carry

    lax.fori_loop(0, BATCH // 16, place_body, 0)

    def do_chunk(c, b, sem):
        pltpu.make_async_copy(tableT_hbm.at[:, pl.ds(0, _CW)], chunkb.at[b],
                              sem).wait()
        start = _scal(off, c)
        end = _scal(cur, c)
        n = end - start
        ng = lax.shift_right_logical(n + 15, 4)

        def group(g2, carry):
            @pl.when(g2 >= 2)
            def _():
                slot = lax.bitwise_and(g2, 1)
                pltpu.make_async_copy(
                    tail_hbm.at[pl.ds(0, 16)],
                    stage.at[pl.ds(slot * 16, 16)], wsem
                ).wait()

            pos = start + g2 * 16
            rv = rbuf[pl.ds(pos, 16)]
            pv = ibuf[pl.ds(pos, 16)]
            col = jnp.clip(rv - lax.shift_left(c, 9), 0, _CW - 1)
            pv = jnp.clip(pv, 0, BATCH - 1)
            slot16 = lax.shift_left(lax.bitwise_and(g2, 1), 4)
            for l in range(16):
                @pl.when(g2 * 16 + l < n)
                def _(l=l):
                    lv = jnp.full((16,), l, jnp.int32)
                    colv = jnp.take(col, lv)
                    bv = jnp.full((16,), b, jnp.int32)
                    for t2 in range(HIDDEN // 16):
                        x = plsc.load_gather(chunkb, [bv, it + t2 * 16, colv])
                        stage[slot16 + l, pl.ds(t2 * 16, 16)] = x
                    ip = jnp.take(pv, lv)[0]
                    pltpu.async_copy(stage.at[slot16 + l], out_hbm.at[ip],
                                     wsem)
            return carry

        lax.fori_loop(0, ng, group, 0)
        k = n - jnp.maximum(ng - 2, 0) * 16

        def dwait(i, carry):
            pltpu.make_async_copy(tail_hbm.at[0], stage.at[0], wsem).wait()
            return carry

        lax.fori_loop(0, k, dwait, 0)

        @pl.when(c + 2 < n_main)
        def _():
            pltpu.async_copy(
                tableT_hbm.at[:, pl.ds(lo + (c + 2) * _CW, _CW)],
                chunkb.at[b], sem,
            )

    def pair_body(cc, carry):
        do_chunk(cc * 2, 0, semA)
        do_chunk(cc * 2 + 1, 1, semB)
        return carry

    lax.fori_loop(0, lax.shift_right_logical(n_main, 1), pair_body, 0)

    @pl.when(lax.bitwise_and(n_main, 1) == 1)
    def _():
        do_chunk(n_main - 1, 0, semA)

    @pl.when(wid == 31)
    def _():
        start = _scal(off, 31)
        end = _scal(cur, 31)
        n = end - start

        lo31 = 31 * _SPAN

        def tail_body(e, carry):
            rr = jnp.clip(_scal(rbuf, start + e) - (_TAIL_LO - lo31), 0, 63)
            ip = jnp.clip(_scal(ibuf, start + e), 0, BATCH - 1)
            pltpu.sync_copy(tail_hbm.at[rr], stage.at[0])
            pltpu.sync_copy(stage.at[0], out_hbm.at[ip])
            return carry

        lax.fori_loop(0, n, tail_body, 0)


def kernel(user_id, table):
    idx = user_id.astype(jnp.int32)
    return _scan_kernel(idx, table.T, table[_TAIL_LO:])


# speedup vs baseline: 2.8544x; 1.0003x over previous
import functools

import jax
import jax.numpy as jnp
from jax import lax
from jax.experimental import pallas as pl
from jax.experimental.pallas import tpu as pltpu
from jax.experimental.pallas import tpu_sc as plsc

HIDDEN = 64
BATCH = 16384
USERS = 1000000

_NW = 32
_CW = 512
_SPAN = 31744
_TAIL_LO = 31 * _SPAN + 31 * _CW
_TRASH = 63

_mesh = plsc.VectorSubcoreMesh(core_axis_name="c", subcore_axis_name="s")
_iota = lambda: lax.iota(jnp.int32, 16)


def _scal(buf, i):
    blk = lax.shift_left(lax.shift_right_logical(i, 4), 4)
    v = buf[pl.ds(blk, 16)]
    return jnp.take(v, jnp.full((16,), lax.bitwise_and(i, 15), jnp.int32))[0]


def _sortseg(cid):
    it = _iota()
    ks, vs = plsc.sort_key_val(cid, it)
    prev = jnp.take(ks, jnp.maximum(it - 1, 0))
    seg_start = (it == 0) | (ks != prev)
    seg_base = plsc.cummax(jnp.where(seg_start, it, 0))
    rank = it - seg_base
    nxt = jnp.take(seg_start.astype(jnp.int32), jnp.minimum(it + 1, 15))
    seg_last = (it == 15) | (nxt == 1)
    return ks, vs, rank, seg_last


@functools.partial(
    pl.kernel,
    mesh=_mesh,
    out_type=jax.ShapeDtypeStruct((BATCH, HIDDEN), jnp.float32),
    scratch_types=[
        pltpu.VMEM((BATCH,), jnp.int32),
        pltpu.VMEM((BATCH + 64,), jnp.int32),
        pltpu.VMEM((BATCH + 64,), jnp.int32),
        pltpu.VMEM((64,), jnp.int32),
        pltpu.VMEM((64,), jnp.int32),
        pltpu.VMEM((64,), jnp.int32),
        pltpu.VMEM((2, HIDDEN, _CW), jnp.float32),
        pltpu.VMEM((32, HIDDEN), jnp.float32),
        pltpu.SemaphoreType.DMA,
        pltpu.SemaphoreType.DMA,
        pltpu.SemaphoreType.DMA,
    ],
    compiler_params=pltpu.CompilerParams(needs_layout_passes=False),
)
def _scan_kernel(idx_hbm, tableT_hbm, tail_hbm, out_hbm, idx_s, rbuf, ibuf,
                 cnt, cur, off, chunkb, stage, semA, semB, wsem):
    wid = lax.axis_index("s") * 2 + lax.axis_index("c")
    lo = wid * _SPAN
    span = jnp.where(wid == 31, USERS - 31 * _SPAN, _SPAN)
    n_main = jnp.where(wid == 31, 31, 62)
    it = _iota()
    zeros = jnp.zeros((16,), jnp.int32)

    def issue_chunk(c, b, sem):
        h = _CW // 2
        pltpu.async_copy(tableT_hbm.at[:, pl.ds(lo + c * _CW, h)],
                         chunkb.at[b, :, pl.ds(0, h)], sem)
        pltpu.async_copy(tableT_hbm.at[:, pl.ds(lo + c * _CW + h, h)],
                         chunkb.at[b, :, pl.ds(h, h)], sem)

    issue_chunk(0, 0, semA)
    issue_chunk(1, 1, semB)
    pltpu.sync_copy(idx_hbm, idx_s)
    for q in range(4):
        cnt[pl.ds(q * 16, 16)] = zeros

    def classify(g):
        iv = idx_s[pl.ds(g * 16, 16)]
        rel = iv - lo
        m = (rel >= 0) & (rel < span)
        cid = jnp.where(m, lax.shift_right_logical(rel, 9), _TRASH)
        return rel, m, cid

    def count_body(g, carry):
        rel, m, cid = classify(g)
        npos = plsc.all_reduce_population_count(m)

        @pl.when(npos[0] > 0)
        def _():
            ks, vs, rank, seg_last = _sortseg(cid)
            plsc.addupdate_scatter(cnt, [ks], rank + 1,
                                   mask=seg_last & (ks != _TRASH))
        return carry

    lax.fori_loop(0, BATCH // 16, count_body, 0)

    carry = jnp.int32(0)
    for q in range(4):
        cv = cnt[pl.ds(q * 16, 16)]
        inc = plsc.cumsum(cv)
        ex = inc - cv + carry
        cur[pl.ds(q * 16, 16)] = ex
        off[pl.ds(q * 16, 16)] = ex
        carry = carry + inc[15]

    def place_body(g, carry):
        rel, m, cid = classify(g)
        npos = plsc.all_reduce_population_count(m)

        @pl.when(npos[0] > 0)
        def _():
            ks, vs, rank, seg_last = _sortseg(cid)
            valid = ks != _TRASH
            slots = plsc.load_gather(cur, [ks]) + rank
            slots = jnp.minimum(slots, BATCH + 48)
            plsc.store_scatter(rbuf, [slots], jnp.take(rel, vs), mask=valid)
            plsc.store_scatter(ibuf, [slots], g * 16 + vs, mask=valid)
            plsc.addupdate_scatter(cur, [ks], rank + 1,
                                   mask=seg_last & valid)
        return carry

    lax.fori_loop(0, BATCH // 16, place_body, 0)

    def do_chunk(c, b, sem):
        pltpu.make_async_copy(tableT_hbm.at[:, pl.ds(0, _CW)], chunkb.at[b],
                              sem).wait()
        start = _scal(off, c)
        end = _scal(cur, c)
        n = end - start
        ng = lax.shift_right_logical(n + 15, 4)

        def group(g2, carry):
            @pl.when(g2 >= 2)
            def _():
                slot = lax.bitwise_and(g2, 1)
                pltpu.make_async_copy(
                    tail_hbm.at[pl.ds(0, 16)],
                    stage.at[pl.ds(slot * 16, 16)], wsem
                ).wait()

            pos = start + g2 * 16
            rv = rbuf[pl.ds(pos, 16)]
            pv = ibuf[pl.ds(pos, 16)]
            col = jnp.clip(rv - lax.shift_left(c, 9), 0, _CW - 1)
            pv = jnp.clip(pv, 0, BATCH - 1)
            slot16 = lax.shift_left(lax.bitwise_and(g2, 1), 4)
            for l in range(16):
                @pl.when(g2 * 16 + l < n)
                def _(l=l):
                    lv = jnp.full((16,), l, jnp.int32)
                    colv = jnp.take(col, lv)
                    bv = jnp.full((16,), b, jnp.int32)
                    for t2 in range(HIDDEN // 16):
                        x = plsc.load_gather(chunkb, [bv, it + t2 * 16, colv])
                        stage[slot16 + l, pl.ds(t2 * 16, 16)] = x
                    ip = jnp.take(pv, lv)[0]
                    pltpu.async_copy(stage.at[slot16 + l], out_hbm.at[ip],
                                     wsem)
            return carry

        lax.fori_loop(0, ng, group, 0)
        k = n - jnp.maximum(ng - 2, 0) * 16

        def dwait(i, carry):
            pltpu.make_async_copy(tail_hbm.at[0], stage.at[0], wsem).wait()
            return carry

        lax.fori_loop(0, k, dwait, 0)

        @pl.when(c + 2 < n_main)
        def _():
            issue_chunk(c + 2, b, sem)

    def pair_body(cc, carry):
        do_chunk(cc * 2, 0, semA)
        do_chunk(cc * 2 + 1, 1, semB)
        return carry

    lax.fori_loop(0, lax.shift_right_logical(n_main, 1), pair_body, 0)

    @pl.when(lax.bitwise_and(n_main, 1) == 1)
    def _():
        do_chunk(n_main - 1, 0, semA)

    @pl.when(wid == 31)
    def _():
        start = _scal(off, 31)
        end = _scal(cur, 31)
        n = end - start

        lo31 = 31 * _SPAN

        def tail_body(e, carry):
            rr = jnp.clip(_scal(rbuf, start + e) - (_TAIL_LO - lo31), 0, 63)
            ip = jnp.clip(_scal(ibuf, start + e), 0, BATCH - 1)
            pltpu.sync_copy(tail_hbm.at[rr], stage.at[0])
            pltpu.sync_copy(stage.at[0], out_hbm.at[ip])
            return carry

        lax.fori_loop(0, n, tail_body, 0)


def kernel(user_id, table):
    idx = user_id.astype(jnp.int32)
    return _scan_kernel(idx, table.T, table[_TAIL_LO:])
